# baseline (speedup 1.0000x reference)
import dataclasses
import functools

import jax
import jax.numpy as jnp
from jax.experimental import pallas as pl
from jax.experimental.pallas import tpu as pltpu
from jax.experimental.pallas import tpu_sc as plsc

LANES = 16
BW = 512
SUB = 128


def _compiler_params():
    cp = pltpu.CompilerParams(use_tc_tiling_on_sc=False)
    if "needs_layout_passes" in pltpu.CompilerParams.__dataclass_fields__:
        cp = dataclasses.replace(cp, needs_layout_passes=False)
    return cp


def kernel(input, tok_table, pos_table):
    batch, seqlen = input.shape
    vocab, hdim = tok_table.shape
    idx2 = input.T.astype(jnp.int32)
    nk = hdim // LANES

    mesh = plsc.VectorSubcoreMesh(core_axis_name="core",
                                  subcore_axis_name="subcore")

    @functools.partial(
        pl.kernel,
        out_type=jax.ShapeDtypeStruct((seqlen, hdim, batch), jnp.float32),
        mesh=mesh,
        compiler_params=_compiler_params(),
        scratch_types=[
            pltpu.VMEM((BW, hdim), jnp.float32),
            pltpu.SemaphoreType.DMA,
        ],
    )
    def emb(tok_hbm, idx_hbm, pos_hbm, out_hbm, tokbuf, sem):
        def body(i_vmem, pos_blk, o_vmem):
            copies = [
                pltpu.async_copy(
                    tok_hbm.at[i_vmem.at[0, pl.ds(s * SUB, SUB)]],
                    tokbuf.at[pl.ds(s * SUB, SUB)],
                    sem,
                )
                for s in range(BW // SUB)
            ]

            pos_vecs = [pos_blk[0, pl.ds(k * LANES, LANES)] for k in range(nk)]
            iota = jnp.arange(LANES, dtype=jnp.int32)
            h_ids = [iota + k * LANES for k in range(nk)]
            skews = [jnp.arange(j, j + LANES, dtype=jnp.int32) % LANES
                     for j in range(LANES)]

            for s in range(BW // SUB):
                copies[s].wait()

                @pl.loop(s * SUB, (s + 1) * SUB, step=2 * LANES)
                def _(r0):
                    r0v = jnp.zeros((LANES,), jnp.int32) + r0
                    for t in range(2):
                        cols = [r0v + (skews[j] + t * LANES) for j in range(LANES)]
                        for k in range(nk):
                            for j in range(LANES):
                                x = plsc.load_gather(tokbuf, [cols[j], h_ids[k]])
                                plsc.store_scatter(o_vmem.at[0],
                                                   [h_ids[k], cols[j]],
                                                   x + pos_vecs[k])

        pltpu.emit_pipeline(
            body,
            grid=(seqlen, batch // BW),
            in_specs=[
                pl.BlockSpec((1, BW), lambda l, c: (l, c)),
                pl.BlockSpec((1, hdim), lambda l, c: (l + 1, 0)),
            ],
            out_specs=[pl.BlockSpec((1, hdim, BW), lambda l, c: (l, 0, c))],
            core_axis_name=("core", "subcore"),
            dimension_semantics=(pltpu.PARALLEL, pltpu.PARALLEL),
        )(idx_hbm, pos_hbm, out_hbm)

    out = emb(tok_table, idx2, pos_table)
    return jnp.transpose(out, (2, 0, 1))


# speedup vs baseline: 1.1990x; 1.1990x over previous
import dataclasses
import functools

import jax
import jax.numpy as jnp
from jax.experimental import pallas as pl
from jax.experimental.pallas import tpu as pltpu
from jax.experimental.pallas import tpu_sc as plsc

LANES = 16
WIN = 800
SUB = 80


def _compiler_params():
    cp = pltpu.CompilerParams(use_tc_tiling_on_sc=False)
    if "needs_layout_passes" in pltpu.CompilerParams.__dataclass_fields__:
        cp = dataclasses.replace(cp, needs_layout_passes=False)
    return cp


def kernel(input, tok_table, pos_table):
    batch, seqlen = input.shape
    vocab, hdim = tok_table.shape
    n = batch * seqlen
    idx = input.reshape(n).astype(jnp.int32)
    nk = hdim // LANES

    mesh = plsc.VectorSubcoreMesh(core_axis_name="core",
                                  subcore_axis_name="subcore")

    @functools.partial(
        pl.kernel,
        out_type=jax.ShapeDtypeStruct((n, hdim), jnp.float32),
        mesh=mesh,
        compiler_params=_compiler_params(),
        scratch_types=[
            pltpu.VMEM((pos_table.shape[0], hdim), jnp.float32),
            pltpu.SemaphoreType.DMA,
        ],
    )
    def emb(tok_hbm, idx_hbm, pos_hbm, out_hbm, pos_vmem, sem):
        pltpu.sync_copy(pos_hbm, pos_vmem)

        nsub = WIN // SUB
        phase = nsub // 2
        jper = WIN // seqlen
        jhalf = jper // 2

        def body(i_vmem, o_vmem):
            copies = [
                pltpu.async_copy(
                    tok_hbm.at[i_vmem.at[pl.ds(s * SUB, SUB)]],
                    o_vmem.at[pl.ds(s * SUB, SUB)],
                    sem,
                )
                for s in range(nsub)
            ]

            for half in range(2):
                for s in range(half * phase, (half + 1) * phase):
                    copies[s].wait()

                @pl.loop(0, seqlen)
                def _(l):
                    pos_vecs = [pos_vmem[l + 1, pl.ds(k * LANES, LANES)]
                                for k in range(nk)]
                    for j in range(half * jhalf, (half + 1) * jhalf):
                        for k in range(nk):
                            plsc.addupdate(
                                o_vmem.at[j * seqlen + l,
                                          pl.ds(k * LANES, LANES)],
                                pos_vecs[k])

        pltpu.emit_pipeline(
            body,
            grid=(n // WIN,),
            in_specs=[pl.BlockSpec((WIN,), lambda i: (i,))],
            out_specs=[pl.BlockSpec((WIN, hdim), lambda i: (i, 0))],
            core_axis_name=("core", "subcore"),
            dimension_semantics=(pltpu.PARALLEL,),
        )(idx_hbm, out_hbm)

    out = emb(tok_table, idx, pos_table)
    return out.reshape(batch, seqlen, hdim)


# speedup vs baseline: 1.2208x; 1.0182x over previous
import dataclasses
import functools

import jax
import jax.numpy as jnp
from jax import lax
from jax.experimental import pallas as pl
from jax.experimental.pallas import tpu as pltpu
from jax.experimental.pallas import tpu_sc as plsc

LANES = 16
WIN = 800
SUB = 80
NWORK = 32


def _compiler_params():
    cp = pltpu.CompilerParams(use_tc_tiling_on_sc=False)
    if "needs_layout_passes" in pltpu.CompilerParams.__dataclass_fields__:
        cp = dataclasses.replace(cp, needs_layout_passes=False)
    return cp


def kernel(input, tok_table, pos_table):
    batch, seqlen = input.shape
    vocab, hdim = tok_table.shape
    n = batch * seqlen
    idx = input.reshape(n).astype(jnp.int32)
    nk = hdim // LANES
    nsub = WIN // SUB
    nchunk = n // (NWORK * WIN)
    jper = WIN // seqlen

    mesh = plsc.VectorSubcoreMesh(core_axis_name="core",
                                  subcore_axis_name="subcore")

    @functools.partial(
        pl.kernel,
        out_type=jax.ShapeDtypeStruct((n, hdim), jnp.float32),
        mesh=mesh,
        compiler_params=_compiler_params(),
        scratch_types=[
            pltpu.VMEM((pos_table.shape[0], hdim), jnp.float32),
            pltpu.VMEM((2, WIN, hdim), jnp.float32),
            pltpu.VMEM((2, WIN), jnp.int32),
            pltpu.SemaphoreType.DMA,
            pltpu.SemaphoreType.DMA,
            pltpu.SemaphoreType.DMA,
        ],
    )
    def emb(tok_hbm, idx_hbm, pos_hbm, out_hbm, pos_vmem, obuf, ibuf,
            sem_g0, sem_g1, sem_out):
        wid = lax.axis_index("subcore") * 2 + lax.axis_index("core")
        base = wid * (nchunk * WIN)
        sems = [sem_g0, sem_g1]

        pltpu.sync_copy(pos_hbm, pos_vmem)

        def fire(c):
            b = c % 2
            pltpu.sync_copy(idx_hbm.at[pl.ds(base + c * WIN, WIN)], ibuf.at[b])
            return [
                pltpu.async_copy(
                    tok_hbm.at[ibuf.at[b, pl.ds(s * SUB, SUB)]],
                    obuf.at[b, pl.ds(s * SUB, SUB)],
                    sems[b],
                )
                for s in range(nsub)
            ]

        gathers = {0: fire(0)}
        writes = {}
        for c in range(nchunk):
            b = c % 2
            if c + 1 < nchunk:
                if c >= 1:
                    writes[c - 1].wait()
                gathers[c + 1] = fire(c + 1)
            for g in gathers.pop(c):
                g.wait()

            @pl.loop(0, seqlen)
            def _(l):
                pos_vecs = [pos_vmem[l + 1, pl.ds(k * LANES, LANES)]
                            for k in range(nk)]
                for j in range(jper):
                    for k in range(nk):
                        plsc.addupdate(
                            obuf.at[b, j * seqlen + l,
                                    pl.ds(k * LANES, LANES)],
                            pos_vecs[k])

            writes[c] = pltpu.async_copy(
                obuf.at[b], out_hbm.at[pl.ds(base + c * WIN, WIN)], sem_out)

        writes[nchunk - 2].wait()
        writes[nchunk - 1].wait()

    out = emb(tok_table, idx, pos_table)
    return out.reshape(batch, seqlen, hdim)
